# Initial kernel scaffold; baseline (speedup 1.0000x reference)
import jax
import jax.numpy as jnp
from jax.experimental import pallas as pl


def kernel(idx, target, table):
    raise NotImplementedError("write your pallas kernel here")


# speedup vs baseline: 1.6954x; 1.6954x over previous
import jax
import jax.numpy as jnp
from jax import lax
from jax.experimental import pallas as pl
from jax.experimental.pallas import tpu as pltpu
from jax.experimental.pallas import tpu_sc as plsc

V = 1000
N = 1024 * 50
NW = 32
NT = N // NW
C = 32
G = NT // C


def _lse_body(tab_ref, lse_ref):
    x = tab_ref[...]
    m = jnp.max(x, axis=1, keepdims=True)
    s = jnp.sum(jnp.exp(x - m), axis=1, keepdims=True)
    lse_ref[...] = m + jnp.log(s)


def _reduce_body(part_ref, out_ref):
    out_ref[...] = (jnp.sum(part_ref[...]) * (1.0 / N)).reshape(1, 1)


def _sc_body(idx_hbm, tgt_hbm, table_hbm, lse_hbm,
             out_hbm, part_hbm,
             idxb0, idxb1, tgtb0, tgtb1, rows0, rows1, lse_v, accv,
             gsem0, gsem1, ssem0, ssem1):
    wid = lax.axis_index("s") * 2 + lax.axis_index("c")
    base = wid * NT

    pltpu.sync_copy(lse_hbm, lse_v)

    pltpu.sync_copy(idx_hbm.at[pl.ds(base, C)], idxb0)
    pltpu.sync_copy(tgt_hbm.at[pl.ds(base, C)], tgtb0)
    pltpu.async_copy(table_hbm.at[idxb0], rows0, gsem0)
    pltpu.sync_copy(idx_hbm.at[pl.ds(base + C, C)], idxb1)
    pltpu.sync_copy(tgt_hbm.at[pl.ds(base + C, C)], tgtb1)
    pltpu.async_copy(table_hbm.at[idxb1], rows1, gsem1)

    zeros16 = jnp.zeros((16,), jnp.int32)
    lanes = lax.iota(jnp.int32, 16)

    def chunk_terms(rows_b, idxb, tgtb, acc):
        for j in range(C // 16):
            sl = pl.ds(j * 16, 16)
            iv = idxb[sl]
            tv = tgtb[sl]
            tgt_logit = plsc.load_gather(rows_b, [lanes + (j * 16), tv])
            lse_g = plsc.load_gather(lse_v, [iv, zeros16])
            acc = acc + (lse_g - tgt_logit)
        return acc

    def main_body(i, acc):
        c0 = 2 * i
        c1 = 2 * i + 1
        pltpu.make_async_copy(table_hbm.at[idxb0], rows0, gsem0).wait()
        pltpu.async_copy(rows0, out_hbm.at[pl.ds(base + c0 * C, C)], ssem0)
        acc = chunk_terms(rows0, idxb0, tgtb0, acc)
        pltpu.make_async_copy(table_hbm.at[idxb1], rows1, gsem1).wait()
        pltpu.async_copy(rows1, out_hbm.at[pl.ds(base + c1 * C, C)], ssem1)
        acc = chunk_terms(rows1, idxb1, tgtb1, acc)
        pltpu.make_async_copy(rows0, out_hbm.at[pl.ds(base + c0 * C, C)],
                              ssem0).wait()

        @pl.when(c0 + 2 < G)
        def _():
            pltpu.sync_copy(idx_hbm.at[pl.ds(base + (c0 + 2) * C, C)], idxb0)
            pltpu.sync_copy(tgt_hbm.at[pl.ds(base + (c0 + 2) * C, C)], tgtb0)
            pltpu.async_copy(table_hbm.at[idxb0], rows0, gsem0)

        pltpu.make_async_copy(rows1, out_hbm.at[pl.ds(base + c1 * C, C)],
                              ssem1).wait()

        @pl.when(c1 + 2 < G)
        def _():
            pltpu.sync_copy(idx_hbm.at[pl.ds(base + (c1 + 2) * C, C)], idxb1)
            pltpu.sync_copy(tgt_hbm.at[pl.ds(base + (c1 + 2) * C, C)], tgtb1)
            pltpu.async_copy(table_hbm.at[idxb1], rows1, gsem1)

        return acc

    acc = lax.fori_loop(0, G // 2, main_body,
                        jnp.zeros((16,), jnp.float32))
    accv[...] = acc
    pltpu.sync_copy(accv, part_hbm.at[wid])


def kernel(idx, target, table):
    idx_f = idx.reshape(-1).astype(jnp.int32)
    tgt_f = target.reshape(-1).astype(jnp.int32)
    table = table.astype(jnp.float32)

    lse = pl.pallas_call(
        _lse_body,
        out_shape=jax.ShapeDtypeStruct((V, 1), jnp.float32),
    )(table)

    sc_call = pl.kernel(
        _sc_body,
        out_type=(
            jax.ShapeDtypeStruct((N, V), jnp.float32),
            jax.ShapeDtypeStruct((NW, 16), jnp.float32),
        ),
        mesh=plsc.VectorSubcoreMesh(core_axis_name="c", subcore_axis_name="s"),
        compiler_params=pltpu.CompilerParams(use_tc_tiling_on_sc=False,
                                             needs_layout_passes=False),
        scratch_types=[
            pltpu.VMEM((C,), jnp.int32),
            pltpu.VMEM((C,), jnp.int32),
            pltpu.VMEM((C,), jnp.int32),
            pltpu.VMEM((C,), jnp.int32),
            pltpu.VMEM((C, V), jnp.float32),
            pltpu.VMEM((C, V), jnp.float32),
            pltpu.VMEM((V, 1), jnp.float32),
            pltpu.VMEM((16,), jnp.float32),
            pltpu.SemaphoreType.DMA,
            pltpu.SemaphoreType.DMA,
            pltpu.SemaphoreType.DMA,
            pltpu.SemaphoreType.DMA,
        ],
    )
    logits2, part = sc_call(idx_f, tgt_f, table, lse)

    loss = pl.pallas_call(
        _reduce_body,
        out_shape=jax.ShapeDtypeStruct((1, 1), jnp.float32),
    )(part)

    return logits2, loss.reshape(())


# speedup vs baseline: 2.5572x; 1.5083x over previous
import jax
import jax.numpy as jnp
from jax import lax
from jax.experimental import pallas as pl
from jax.experimental.pallas import tpu as pltpu
from jax.experimental.pallas import tpu_sc as plsc

V = 1000
VP = 1024
N = 1024 * 50
NW = 32
NT = N // NW
C = 32
G = NT // C
LC = 80
LG = NT // LC


def _lse_body(tab_ref, lse_ref):
    x = tab_ref[...]
    m = jnp.max(x, axis=1, keepdims=True)
    s = jnp.sum(jnp.exp(x - m), axis=1, keepdims=True)
    lse_ref[...] = m + jnp.log(s)


def _reduce_body(part_ref, out_ref):
    out_ref[...] = (jnp.sum(part_ref[...]) * (1.0 / N)).reshape(1, 1)


def _sc_loss_body(idx_hbm, tgt_hbm, tabf_hbm, lse_hbm,
                  part_hbm,
                  idxc, tgtc, flatc, valc, lse_v, accv, psem):
    wid = lax.axis_index("s") * 2 + lax.axis_index("c")
    base = wid * NT

    pltpu.sync_copy(lse_hbm, lse_v)
    zeros16 = jnp.zeros((16,), jnp.int32)

    def loss_body(k, acc):
        off = base + k * LC
        pltpu.sync_copy(idx_hbm.at[pl.ds(off, LC)], idxc)
        pltpu.sync_copy(tgt_hbm.at[pl.ds(off, LC)], tgtc)
        for j in range(LC // 16):
            sl = pl.ds(j * 16, 16)
            flatc[sl] = idxc[sl] * V + tgtc[sl]
        pltpu.async_copy(tabf_hbm.at[flatc], valc, psem).wait()
        for j in range(LC // 16):
            sl = pl.ds(j * 16, 16)
            lse_g = plsc.load_gather(lse_v, [idxc[sl], zeros16])
            acc = acc + (lse_g - valc[sl])
        return acc

    acc = lax.fori_loop(0, LG, loss_body, jnp.zeros((16,), jnp.float32))
    accv[...] = acc
    pltpu.sync_copy(accv, part_hbm.at[wid])


def _sc_gather_body(idx_hbm, table_hbm, out_hbm,
                    idxb0, idxb1, rows0, rows1,
                    gsem0, gsem1, ssem0, ssem1):
    wid = lax.axis_index("s") * 2 + lax.axis_index("c")
    base = wid * NT

    pltpu.sync_copy(idx_hbm.at[pl.ds(base, C)], idxb0)
    pltpu.async_copy(table_hbm.at[idxb0], rows0, gsem0)
    pltpu.sync_copy(idx_hbm.at[pl.ds(base + C, C)], idxb1)
    pltpu.async_copy(table_hbm.at[idxb1], rows1, gsem1)

    def main_body(i, carry):
        c0 = 2 * i
        c1 = 2 * i + 1
        pltpu.make_async_copy(table_hbm.at[idxb0], rows0, gsem0).wait()
        pltpu.async_copy(rows0, out_hbm.at[pl.ds(base + c0 * C, C)], ssem0)
        pltpu.make_async_copy(table_hbm.at[idxb1], rows1, gsem1).wait()
        pltpu.async_copy(rows1, out_hbm.at[pl.ds(base + c1 * C, C)], ssem1)
        pltpu.make_async_copy(rows0, out_hbm.at[pl.ds(base + c0 * C, C)],
                              ssem0).wait()

        @pl.when(c0 + 2 < G)
        def _():
            pltpu.sync_copy(idx_hbm.at[pl.ds(base + (c0 + 2) * C, C)], idxb0)
            pltpu.async_copy(table_hbm.at[idxb0], rows0, gsem0)

        pltpu.make_async_copy(rows1, out_hbm.at[pl.ds(base + c1 * C, C)],
                              ssem1).wait()

        @pl.when(c1 + 2 < G)
        def _():
            pltpu.sync_copy(idx_hbm.at[pl.ds(base + (c1 + 2) * C, C)], idxb1)
            pltpu.async_copy(table_hbm.at[idxb1], rows1, gsem1)

        return carry

    lax.fori_loop(0, G // 2, main_body, 0)


_MESH = dict(core_axis_name="c", subcore_axis_name="s")


def kernel(idx, target, table):
    idx_f = idx.reshape(-1).astype(jnp.int32)
    tgt_f = target.reshape(-1).astype(jnp.int32)
    table = table.astype(jnp.float32)

    lse = pl.pallas_call(
        _lse_body,
        out_shape=jax.ShapeDtypeStruct((V, 1), jnp.float32),
    )(table)

    loss_call = pl.kernel(
        _sc_loss_body,
        out_type=jax.ShapeDtypeStruct((NW, 16), jnp.float32),
        mesh=plsc.VectorSubcoreMesh(**_MESH),
        compiler_params=pltpu.CompilerParams(use_tc_tiling_on_sc=False,
                                             needs_layout_passes=False),
        scratch_types=[
            pltpu.VMEM((LC,), jnp.int32),
            pltpu.VMEM((LC,), jnp.int32),
            pltpu.VMEM((LC,), jnp.int32),
            pltpu.VMEM((LC,), jnp.float32),
            pltpu.VMEM((V, 1), jnp.float32),
            pltpu.VMEM((16,), jnp.float32),
            pltpu.SemaphoreType.DMA,
        ],
    )
    part = loss_call(idx_f, tgt_f, table.reshape(-1), lse)

    gather_call = pl.kernel(
        _sc_gather_body,
        out_type=jax.ShapeDtypeStruct((N, VP), jnp.float32),
        mesh=plsc.VectorSubcoreMesh(**_MESH),
        compiler_params=pltpu.CompilerParams(needs_layout_passes=False),
        scratch_types=[
            pltpu.VMEM((C,), jnp.int32),
            pltpu.VMEM((C,), jnp.int32),
            pltpu.VMEM((C, VP), jnp.float32),
            pltpu.VMEM((C, VP), jnp.float32),
            pltpu.SemaphoreType.DMA,
            pltpu.SemaphoreType.DMA,
            pltpu.SemaphoreType.DMA,
            pltpu.SemaphoreType.DMA,
        ],
    )
    table_pad = jnp.pad(table, ((0, 0), (0, VP - V)))
    out_pad = gather_call(idx_f, table_pad)
    logits2 = out_pad[:, :V]

    loss = pl.pallas_call(
        _reduce_body,
        out_shape=jax.ShapeDtypeStruct((1, 1), jnp.float32),
    )(part)

    return logits2, loss.reshape(())
